# baseline (speedup 1.0000x reference)
import functools

import jax
import jax.numpy as jnp
from jax import lax
from jax.experimental import pallas as pl
from jax.experimental.pallas import tpu as pltpu
from jax.experimental.pallas import tpu_sc as plsc

_N, _K = 16384, 26
_NW = 32
_SC_COLS = 3072
_ACT = _SC_COLS // 128
_BC = 1024
_CPT = 128
_NT = _CPT // 16

_LN2 = 0.6931471805599453
_C0 = 0.0009250321113061233 - 127.0 * _LN2
_C1 = 0.9735508519008734
_C2 = -0.3921667221516742
_C3 = 0.11255014928628229


def _log_term(x):
    x = jnp.minimum(jnp.maximum(x, 0.01), 0.99)
    bits = lax.bitcast_convert_type(x, jnp.int32)
    eb = bits >> 23
    m = (bits & 0x7FFFFF) | 0x3F800000
    u = lax.bitcast_convert_type(m, jnp.float32) - 1.0
    r = _C3
    r = r * u + _C2
    r = r * u + _C1
    r = r * u + _C0
    return eb.astype(jnp.float32) * _LN2 + r


_mesh = plsc.VectorSubcoreMesh(core_axis_name="c", subcore_axis_name="s")


@functools.partial(
    pl.kernel,
    mesh=_mesh,
    out_type=jax.ShapeDtypeStruct((2, _NW, 16), jnp.float32),
    scratch_types=[
        pltpu.VMEM((_K, _CPT), jnp.float32),
        pltpu.VMEM((_K, _CPT), jnp.float32),
        pltpu.VMEM((_K, _CPT), jnp.float32),
        pltpu.VMEM((_K, _CPT), jnp.float32),
        pltpu.VMEM((16,), jnp.float32),
        pltpu.SemaphoreType.DMA,
        pltpu.SemaphoreType.DMA,
        pltpu.SemaphoreType.DMA,
        pltpu.SemaphoreType.DMA,
    ],
)
def _sc_loss(p1, p2, q1, q2, out, a_v, b_v, c_v, d_v, acc_v,
             s1, s2, s3, s4):
    wid = lax.axis_index("s") * 2 + lax.axis_index("c")
    active = wid < _ACT
    col0 = jnp.minimum(wid, _ACT - 1) * _CPT
    cp1 = pltpu.async_copy(p1.at[:, pl.ds(col0, _CPT)], a_v, s1)
    cp2 = pltpu.async_copy(q2.at[:, pl.ds(col0, _CPT)], b_v, s2)
    cp3 = pltpu.async_copy(p2.at[:, pl.ds(col0, _CPT)], c_v, s3)
    cp4 = pltpu.async_copy(q1.at[:, pl.ds(col0, _CPT)], d_v, s4)

    zero = jnp.zeros((16,), jnp.float32)
    nacc = 8

    def make_body(pr_v, po_v):
        def body(j, accs):
            accs = list(accs)
            c = j * 16
            for r in range(_K):
                x = pr_v[r, pl.ds(c, 16)]
                w = po_v[r, pl.ds(c, 16)]
                accs[r % nacc] = accs[r % nacc] + w * _log_term(x)
            return tuple(accs)
        return body

    cp1.wait()
    cp2.wait()
    accs = lax.fori_loop(0, _NT, make_body(a_v, b_v), (zero,) * nacc)
    acc1 = ((accs[0] + accs[1]) + (accs[2] + accs[3])) + \
           ((accs[4] + accs[5]) + (accs[6] + accs[7]))

    cp3.wait()
    cp4.wait()
    accs = lax.fori_loop(0, _NT, make_body(c_v, d_v), (zero,) * nacc)
    acc2 = ((accs[0] + accs[1]) + (accs[2] + accs[3])) + \
           ((accs[4] + accs[5]) + (accs[6] + accs[7]))

    zmask = jnp.where(active, 1.0, 0.0)
    acc_v[...] = acc1 * zmask
    pltpu.sync_copy(acc_v, out.at[0, wid])
    acc_v[...] = acc2 * zmask
    pltpu.sync_copy(acc_v, out.at[1, wid])


def _tc_body(p1, p2, q1, q2, out_ref):
    i = pl.program_id(0)

    @pl.when(i == 0)
    def _init():
        out_ref[...] = jnp.zeros_like(out_ref)

    def contrib(pr, po):
        x = jnp.clip(pr[...], 0.01, 0.99) + 1e-10
        return jnp.sum(po[...] * jnp.log(x), axis=1)

    out_ref[0, :] += contrib(p1, q2)
    out_ref[1, :] += contrib(p2, q1)


def _tc_loss(p1t, p2t, q1t, q2t):
    nblk = (_N - _SC_COLS) // _BC
    spec = pl.BlockSpec((_K, _BC), lambda i: (0, i + _SC_COLS // _BC))
    return pl.pallas_call(
        _tc_body, grid=(nblk,),
        in_specs=[spec, spec, spec, spec],
        out_specs=pl.BlockSpec((2, _K), lambda i: (0, 0)),
        out_shape=jax.ShapeDtypeStruct((2, _K), jnp.float32),
    )(p1t, p2t, q1t, q2t)


def kernel(prior_1, prior_2, post_1, post_2):
    t = (prior_1.T, prior_2.T, post_1.T, post_2.T)
    parts_tc = _tc_loss(*t)
    parts_sc = _sc_loss(*t)
    s = jnp.sum(parts_sc, axis=(1, 2)) + jnp.sum(parts_tc, axis=1)
    losses = 0.05 - s / _N
    return (losses[0], losses[1])


# speedup vs baseline: 1.0817x; 1.0817x over previous
import functools

import jax
import jax.numpy as jnp
from jax import lax
from jax.experimental import pallas as pl
from jax.experimental.pallas import tpu as pltpu
from jax.experimental.pallas import tpu_sc as plsc

_N, _K = 16384, 26
_NW = 32
_SC_COLS = 4096
_BC = 2048
_CPT = _SC_COLS // _NW
_NT = _CPT // 16

_LN2 = 0.6931471805599453
_C0 = 0.0009250321113061233 - 127.0 * _LN2
_C1 = 0.9735508519008734
_C2 = -0.3921667221516742
_C3 = 0.11255014928628229


def _log_term(x):
    x = jnp.minimum(jnp.maximum(x, 0.01), 0.99)
    bits = lax.bitcast_convert_type(x, jnp.int32)
    eb = bits >> 23
    m = (bits & 0x7FFFFF) | 0x3F800000
    u = lax.bitcast_convert_type(m, jnp.float32) - 1.0
    r = _C3
    r = r * u + _C2
    r = r * u + _C1
    r = r * u + _C0
    return eb.astype(jnp.float32) * _LN2 + r


_mesh = plsc.VectorSubcoreMesh(core_axis_name="c", subcore_axis_name="s")


@functools.partial(
    pl.kernel,
    mesh=_mesh,
    out_type=jax.ShapeDtypeStruct((2, _NW, 16), jnp.float32),
    scratch_types=[
        pltpu.VMEM((_K, _CPT), jnp.float32),
        pltpu.VMEM((_K, _CPT), jnp.float32),
        pltpu.VMEM((_K, _CPT), jnp.float32),
        pltpu.VMEM((_K, _CPT), jnp.float32),
        pltpu.VMEM((16,), jnp.float32),
        pltpu.SemaphoreType.DMA,
        pltpu.SemaphoreType.DMA,
        pltpu.SemaphoreType.DMA,
        pltpu.SemaphoreType.DMA,
    ],
)
def _sc_loss(p1, p2, q1, q2, out, a_v, b_v, c_v, d_v, acc_v,
             s1, s2, s3, s4):
    wid = lax.axis_index("s") * 2 + lax.axis_index("c")
    col0 = wid * _CPT
    cp1 = pltpu.async_copy(p1.at[:, pl.ds(col0, _CPT)], a_v, s1)
    cp2 = pltpu.async_copy(q2.at[:, pl.ds(col0, _CPT)], b_v, s2)
    cp3 = pltpu.async_copy(p2.at[:, pl.ds(col0, _CPT)], c_v, s3)
    cp4 = pltpu.async_copy(q1.at[:, pl.ds(col0, _CPT)], d_v, s4)

    zero = jnp.zeros((16,), jnp.float32)
    nacc = 8

    def make_body(pr_v, po_v):
        def body(j, accs):
            accs = list(accs)
            c = j * 16
            for r in range(_K):
                x = pr_v[r, pl.ds(c, 16)]
                w = po_v[r, pl.ds(c, 16)]
                accs[r % nacc] = accs[r % nacc] + w * _log_term(x)
            return tuple(accs)
        return body

    cp1.wait()
    cp2.wait()
    accs = lax.fori_loop(0, _NT, make_body(a_v, b_v), (zero,) * nacc)
    acc1 = ((accs[0] + accs[1]) + (accs[2] + accs[3])) + \
           ((accs[4] + accs[5]) + (accs[6] + accs[7]))

    cp3.wait()
    cp4.wait()
    accs = lax.fori_loop(0, _NT, make_body(c_v, d_v), (zero,) * nacc)
    acc2 = ((accs[0] + accs[1]) + (accs[2] + accs[3])) + \
           ((accs[4] + accs[5]) + (accs[6] + accs[7]))

    acc_v[...] = acc1
    pltpu.sync_copy(acc_v, out.at[0, wid])
    acc_v[...] = acc2
    pltpu.sync_copy(acc_v, out.at[1, wid])


def _tc_body(p1, p2, q1, q2, out_ref):
    i = pl.program_id(0)

    @pl.when(i == 0)
    def _init():
        out_ref[...] = jnp.zeros_like(out_ref)

    def contrib(pr, po):
        x = jnp.clip(pr[...], 0.01, 0.99) + 1e-10
        return jnp.sum(po[...] * jnp.log(x), axis=1)

    out_ref[0, :] += contrib(p1, q2)
    out_ref[1, :] += contrib(p2, q1)


def _tc_loss(p1t, p2t, q1t, q2t):
    nblk = (_N - _SC_COLS) // _BC
    spec = pl.BlockSpec((_K, _BC), lambda i: (0, i + _SC_COLS // _BC))
    return pl.pallas_call(
        _tc_body, grid=(nblk,),
        in_specs=[spec, spec, spec, spec],
        out_specs=pl.BlockSpec((2, _K), lambda i: (0, 0)),
        out_shape=jax.ShapeDtypeStruct((2, _K), jnp.float32),
    )(p1t, p2t, q1t, q2t)


def kernel(prior_1, prior_2, post_1, post_2):
    t = (prior_1.T, prior_2.T, post_1.T, post_2.T)
    parts_tc = _tc_loss(*t)
    parts_sc = _sc_loss(*t)
    s = jnp.sum(parts_sc, axis=(1, 2)) + jnp.sum(parts_tc, axis=1)
    losses = 0.05 - s / _N
    return (losses[0], losses[1])


# speedup vs baseline: 1.1146x; 1.0304x over previous
import functools

import jax
import jax.numpy as jnp
from jax import lax
from jax.experimental import pallas as pl
from jax.experimental.pallas import tpu as pltpu
from jax.experimental.pallas import tpu_sc as plsc

_N, _K = 16384, 26
_NW = 32
_SC_COLS = 4096
_BC = 2048
_CPT = _SC_COLS // _NW
_NT = _CPT // 16

_LN2 = 0.6931471805599453
_C0 = 0.0009250321113061233 - 127.0 * _LN2
_C1 = 0.9735508519008734
_C2 = -0.3921667221516742
_C3 = 0.11255014928628229


def _log_term(x):
    x = jnp.minimum(jnp.maximum(x, 0.01), 0.99)
    bits = lax.bitcast_convert_type(x, jnp.int32)
    eb = bits >> 23
    m = (bits & 0x7FFFFF) | 0x3F800000
    u = lax.bitcast_convert_type(m, jnp.float32) - 1.0
    r = _C3
    r = r * u + _C2
    r = r * u + _C1
    r = r * u + _C0
    return eb.astype(jnp.float32) * _LN2 + r


_mesh = plsc.VectorSubcoreMesh(core_axis_name="c", subcore_axis_name="s")


@functools.partial(
    pl.kernel,
    mesh=_mesh,
    out_type=jax.ShapeDtypeStruct((2, _NW, 16), jnp.float32),
    scratch_types=[
        pltpu.VMEM((2, _K, _CPT), jnp.float32),
        pltpu.VMEM((2, _K, _CPT), jnp.float32),
        pltpu.VMEM((16,), jnp.float32),
        pltpu.SemaphoreType.DMA,
        pltpu.SemaphoreType.DMA,
        pltpu.SemaphoreType.DMA,
        pltpu.SemaphoreType.DMA,
    ],
)
def _sc_loss(p1, p2, q1, q2, out, pr4, po4, acc_v, s1, s2, s3, s4):
    wid = lax.axis_index("s") * 2 + lax.axis_index("c")
    col0 = wid * _CPT
    cp1 = pltpu.async_copy(p1.at[:, pl.ds(col0, _CPT)], pr4.at[0], s1)
    cp2 = pltpu.async_copy(q2.at[:, pl.ds(col0, _CPT)], po4.at[0], s2)
    cp3 = pltpu.async_copy(p2.at[:, pl.ds(col0, _CPT)], pr4.at[1], s3)
    cp4 = pltpu.async_copy(q1.at[:, pl.ds(col0, _CPT)], po4.at[1], s4)
    cp1.wait()
    cp2.wait()
    cp3.wait()
    cp4.wait()

    zero = jnp.zeros((16,), jnp.float32)
    nacc = 8

    def branch(b, _):
        def body(j, accs):
            accs = list(accs)
            c = j * 16
            for r in range(_K):
                x = pr4[b, r, pl.ds(c, 16)]
                w = po4[b, r, pl.ds(c, 16)]
                accs[r % nacc] = accs[r % nacc] + w * _log_term(x)
            return tuple(accs)
        accs = lax.fori_loop(0, _NT, body, (zero,) * nacc)
        acc_v[...] = ((accs[0] + accs[1]) + (accs[2] + accs[3])) + \
                     ((accs[4] + accs[5]) + (accs[6] + accs[7]))
        pltpu.sync_copy(acc_v, out.at[b, wid])
        return 0

    lax.fori_loop(0, 2, branch, 0)


def _tc_body(p1, p2, q1, q2, out_ref):
    i = pl.program_id(0)

    @pl.when(i == 0)
    def _init():
        out_ref[...] = jnp.zeros_like(out_ref)

    def contrib(pr, po):
        x = jnp.clip(pr[...], 0.01, 0.99) + 1e-10
        return jnp.sum(po[...] * jnp.log(x), axis=1)

    out_ref[0, :] += contrib(p1, q2)
    out_ref[1, :] += contrib(p2, q1)


def _tc_loss(p1t, p2t, q1t, q2t):
    nblk = (_N - _SC_COLS) // _BC
    spec = pl.BlockSpec((_K, _BC), lambda i: (0, i + _SC_COLS // _BC))
    return pl.pallas_call(
        _tc_body, grid=(nblk,),
        in_specs=[spec, spec, spec, spec],
        out_specs=pl.BlockSpec((2, _K), lambda i: (0, 0)),
        out_shape=jax.ShapeDtypeStruct((2, _K), jnp.float32),
    )(p1t, p2t, q1t, q2t)


def kernel(prior_1, prior_2, post_1, post_2):
    t = (prior_1.T, prior_2.T, post_1.T, post_2.T)
    parts_tc = _tc_loss(*t)
    parts_sc = _sc_loss(*t)
    s = jnp.sum(parts_sc, axis=(1, 2)) + jnp.sum(parts_tc, axis=1)
    losses = 0.05 - s / _N
    return (losses[0], losses[1])
